# baseline (speedup 1.0000x reference)
import functools

import jax
import jax.numpy as jnp
from jax import lax
from jax.experimental import pallas as pl
from jax.experimental.pallas import tpu as pltpu
from jax.experimental.pallas import tpu_sc as plsc

_N = 10000
_E = 160000
_HID = 133
_FP = 144
_ROW = 2 * _FP
_B = 1024
_NC = 2
_NS = 16
_NW = _NC * _NS
_EPW = _E // _NW
_CH = 20
_NCHUNK = _EPW // _CH
_NZS = 10
_RPS = _N // _NZS


def _edge_body(h_hbm, src_hbm, dst_hbm, a_hbm, z_hbm, out_hbm,
               idx_s, idx_d, hs0, hd0, hs1, hd1, msg0, msg1, acc,
               gs0, gd0, gs1, gd1, ss0, ss1):
    cid = lax.axis_index("c")
    sid = lax.axis_index("s")
    wid = cid * _NS + sid

    pltpu.sync_copy(a_hbm, msg0.at[pl.ds(0, 2)])
    pltpu.sync_copy(src_hbm.at[wid], idx_s)
    pltpu.sync_copy(dst_hbm.at[wid], idx_d)

    a_regs = [msg0[j // 9, pl.ds(16 * (j % 9), 16)] for j in range(18)]
    lane = lax.iota(jnp.int32, 16)
    perms = [lane ^ k for k in (8, 4, 2, 1)]

    def _allsum(v):
        for p in perms:
            v = v + v.at[p].get(mode="promise_in_bounds")
        return v

    bufs = ((hs0, hd0, msg0, gs0, gd0, ss0), (hs1, hd1, msg1, gs1, gd1, ss1))

    def _issue(c, s):
        hs, hd, msg, gs, gd, ss = bufs[s]
        pltpu.async_copy(h_hbm.at[idx_s.at[c]], hs, gs)
        pltpu.async_copy(h_hbm.at[idx_d.at[c]], hd, gd)

    def _step(c, s):
        hs, hd, msg, gs, gd, ss = bufs[s]
        pltpu.make_async_copy(h_hbm.at[idx_s.at[c]], hs, gs).wait()
        pltpu.make_async_copy(h_hbm.at[idx_d.at[c]], hd, gd).wait()

        @pl.when(c >= 2)
        def _():
            pltpu.make_async_copy(msg, acc.at[idx_d.at[c]], ss).wait()

        @plsc.parallel_loop(0, _CH)
        def edge_body(e):
            acc0 = jnp.zeros((16,), jnp.float32)
            acc1 = jnp.zeros((16,), jnp.float32)
            for j in range(18):
                sv = hs[e, pl.ds(16 * j, 16)]
                dv = hd[e, pl.ds(16 * j, 16)]
                z = sv + dv
                zc = jnp.minimum(jnp.maximum(z, -9.1), 9.1)
                e2 = jnp.exp(zc + zc)
                th = (e2 - 1.0) / (e2 + 1.0)
                contrib = th * a_regs[j]
                if j < 9:
                    acc0 = acc0 + contrib
                else:
                    acc1 = acc1 + contrib
            s0 = _allsum(acc0)
            s1 = _allsum(acc1)
            el0 = jnp.where(s0 > 0.0, s0, jnp.exp(s0) - 1.0)
            el1 = jnp.where(s1 > 0.0, s1, jnp.exp(s1) - 1.0)
            att0 = 0.5 / (1.0 + jnp.exp(el1 - el0))
            att1 = 0.5 - att0
            for j in range(9):
                msg[e, pl.ds(16 * j, 16)] = (att0 * hs[e, pl.ds(16 * j, 16)]
                                             + att1 * hs[e, pl.ds(16 * (j + 9), 16)])
        @pl.when(c + 2 < _NCHUNK)
        def _():
            _issue(c + 2, s)

        pltpu.async_copy(msg, acc.at[idx_d.at[c]], ss, add=True)

    _issue(0, 0)
    _issue(1, 1)

    @pl.when(sid < _NZS)
    def _():
        pltpu.sync_copy(z_hbm, acc.at[pl.ds(sid * _RPS, _RPS)])
    plsc.subcore_barrier()

    def pair_body(g, carry):
        _step(2 * g, 0)
        _step(2 * g + 1, 1)
        return carry

    lax.fori_loop(0, _NCHUNK // 2, pair_body, 0)
    pltpu.make_async_copy(msg0, acc.at[idx_d.at[_NCHUNK - 2]], ss0).wait()
    pltpu.make_async_copy(msg1, acc.at[idx_d.at[_NCHUNK - 1]], ss1).wait()
    plsc.subcore_barrier()

    @pl.when(sid < _NZS)
    def _():
        pltpu.sync_copy(acc.at[pl.ds(sid * _RPS, _RPS)],
                        out_hbm.at[pl.ds(cid * _N + sid * _RPS, _RPS)])


def _run_edge_layer(h_pad, src, dst, a_pad, zeros_hbm):
    mesh = plsc.VectorSubcoreMesh(core_axis_name="c", subcore_axis_name="s",
                                  num_cores=_NC, num_subcores=_NS)
    f = pl.kernel(
        _edge_body,
        out_type=jax.ShapeDtypeStruct((2 * _N, _FP), jnp.float32),
        mesh=mesh,
        scratch_types=[
            pltpu.VMEM((_NCHUNK, _CH), jnp.int32),
            pltpu.VMEM((_NCHUNK, _CH), jnp.int32),
            pltpu.VMEM((_CH, _ROW), jnp.float32),
            pltpu.VMEM((_CH, _ROW), jnp.float32),
            pltpu.VMEM((_CH, _ROW), jnp.float32),
            pltpu.VMEM((_CH, _ROW), jnp.float32),
            pltpu.VMEM((_CH, _FP), jnp.float32),
            pltpu.VMEM((_CH, _FP), jnp.float32),
            pltpu.VMEM_SHARED((_N, _FP), jnp.float32),
            pltpu.SemaphoreType.DMA,
            pltpu.SemaphoreType.DMA,
            pltpu.SemaphoreType.DMA,
            pltpu.SemaphoreType.DMA,
            pltpu.SemaphoreType.DMA,
            pltpu.SemaphoreType.DMA,
        ],
        compiler_params=pltpu.CompilerParams(use_tc_tiling_on_sc=False),
    )
    return f(h_pad, src, dst, a_pad, zeros_hbm)


def _mm1_body(x_ref, w_ref, b_ref, o_ref):
    o_ref[...] = (jnp.dot(x_ref[...], w_ref[...],
                          preferred_element_type=jnp.float32) + b_ref[...])


def _mid_body(p_ref, wl_ref, bl_ref, w2_ref, b2_ref, o_ref):
    hsum = p_ref[0:_N, :] + p_ref[_N:2 * _N, :]
    mid = (jnp.dot(hsum, wl_ref[...], preferred_element_type=jnp.float32)
           + bl_ref[...])
    o_ref[...] = (jnp.dot(mid, w2_ref[...], preferred_element_type=jnp.float32)
                  + b2_ref[...])


def _geno_body(g_ref, fgi_ref, wq_ref, bq_ref, wk_ref, bk_ref, wg_ref,
               wp_ref, bp_ref, wf_ref, bf_ref, fgt_ref, geno_ref, fg_ref):
    f32 = jnp.float32
    g = g_ref[...]
    q = jnp.dot(g, wq_ref[...], preferred_element_type=f32) + bq_ref[...]
    qn = q * (1.0 / jnp.maximum(
        jnp.sqrt(jnp.sum(q * q, axis=1, keepdims=True)), 1e-12))
    kk = jnp.dot(g, wk_ref[...], preferred_element_type=f32) + bk_ref[...]
    kn = kk * (1.0 / jnp.maximum(
        jnp.sqrt(jnp.sum(kk * kk, axis=1, keepdims=True)), 1e-12))
    qw = jnp.sum(qn * wg_ref[...], axis=1, keepdims=True)
    v = qw * (_HID ** -0.5)
    A = v / jnp.maximum(jnp.abs(v), 1e-12)
    G = A * qn
    t1 = (jnp.dot(G * kn, wp_ref[...], preferred_element_type=f32)
          + bp_ref[...] + qn)
    geno_ref[...] = (jnp.dot(t1, wf_ref[...], preferred_element_type=f32)
                     + bf_ref[...])

    vi = lax.broadcasted_iota(jnp.int32, (_B, 200), 1)
    cnt = jnp.zeros((_B, 200), f32)
    for j in range(8):
        cnt = cnt + (fgi_ref[:, j:j + 1] == vi).astype(f32)
    fg_ref[...] = jnp.dot(cnt, fgt_ref[...],
                          preferred_element_type=f32) * 0.125


def _final_body(p_ref, geno_ref, fg_ref, wa_ref, wb_ref,
                bc1_ref, wo_ref, bo_ref, o_ref):
    f32 = jnp.float32
    h2 = p_ref[0:_N, :] + p_ref[_N:2 * _N, :]
    drug_vec = jnp.sum(h2, axis=0, keepdims=True) * (1.0 / _N)
    geno = geno_ref[...]
    drug = drug_vec + fg_ref[...]

    mask = (lax.broadcasted_iota(jnp.int32, (1, _FP), 1) < _HID).astype(f32)
    dmean = jnp.sum(drug, axis=1, keepdims=True) * (1.0 / _HID)
    gmean = jnp.sum(geno, axis=1, keepdims=True) * (1.0 / _HID)
    dm = (drug - dmean) * mask
    gm = (geno - gmean) * mask
    num = jnp.sum(dm * gm, axis=1, keepdims=True)
    den = (jnp.sqrt(jnp.sum(dm * dm, axis=1, keepdims=True))
           * jnp.sqrt(jnp.sum(gm * gm, axis=1, keepdims=True)) + 1e-12)
    boost = 1.0 + jax.nn.sigmoid(num / den)

    hc = jnp.maximum(
        jnp.dot(drug * boost, wa_ref[...], preferred_element_type=f32)
        + jnp.dot(geno * boost, wb_ref[...], preferred_element_type=f32)
        + bc1_ref[...], 0.0)
    o_ref[...] = jax.nn.sigmoid(
        jnp.dot(hc, wo_ref[...], preferred_element_type=f32) + bo_ref[...])


def _pad_heads_cols(W):
    out = jnp.zeros((W.shape[0], _ROW), jnp.float32)
    out = out.at[:, 0:_HID].set(W[:, 0:_HID])
    out = out.at[:, _FP:_FP + _HID].set(W[:, _HID:2 * _HID])
    return out


def kernel(x, genomic_feats, W1, b1, a1, Wl, bl, W2, b2, a2, fg_table, Wq, bq,
           Wk, bk, w_g, Wp, bp, Wf, bf, Wc1, bc1, Wo, bo, edge_index,
           fg_indices):
    f32 = jnp.float32
    src = edge_index[0].reshape(_NW, _NCHUNK, _CH)
    dst = edge_index[1].reshape(_NW, _NCHUNK, _CH)

    W1p = _pad_heads_cols(W1)
    b1p = _pad_heads_cols(b1[None, :])
    a1p = _pad_heads_cols(a1.reshape(1, 2 * _HID)).reshape(2, _FP)
    W2p = jnp.zeros((_FP, _ROW), f32).at[:_HID].set(_pad_heads_cols(W2))
    b2p = _pad_heads_cols(b2[None, :])
    a2p = _pad_heads_cols(a2.reshape(1, 2 * _HID)).reshape(2, _FP)
    Wlp = jnp.zeros((_FP, _FP), f32).at[:_HID, :_HID].set(Wl)
    blp = jnp.zeros((1, _FP), f32).at[0, :_HID].set(bl)
    Wq_p = jnp.zeros((1024, _FP), f32).at[:, :_HID].set(Wq)
    bq_p = jnp.zeros((1, _FP), f32).at[0, :_HID].set(bq)
    Wk_p = jnp.zeros((1024, _FP), f32).at[:, :_HID].set(Wk)
    bk_p = jnp.zeros((1, _FP), f32).at[0, :_HID].set(bk)
    wg_p = jnp.zeros((1, _FP), f32).at[0, :_HID].set(w_g[:, 0])
    Wp_p = jnp.zeros((_FP, _FP), f32).at[:_HID, :_HID].set(Wp)
    bp_p = jnp.zeros((1, _FP), f32).at[0, :_HID].set(bp)
    Wf_p = jnp.zeros((_FP, _FP), f32).at[:_HID, :_HID].set(Wf)
    bf_p = jnp.zeros((1, _FP), f32).at[0, :_HID].set(bf)
    fgt_p = jnp.zeros((200, _FP), f32).at[:, :_HID].set(fg_table)
    Wc1a = jnp.zeros((_FP, 128), f32).at[:_HID].set(Wc1[:_HID])
    Wc1b = jnp.zeros((_FP, 128), f32).at[:_HID].set(Wc1[_HID:])
    Wo_pad = jnp.zeros((128, 128), f32).at[:, :1].set(Wo)
    bo_pad = jnp.zeros((1, 128), f32).at[:, :1].set(bo[None, :])
    zeros_hbm = jnp.zeros((_RPS, _FP), f32)

    h1tab = pl.pallas_call(
        _mm1_body,
        out_shape=jax.ShapeDtypeStruct((_N, _ROW), f32),
    )(x, W1p, b1p)

    geno, fg = pl.pallas_call(
        _geno_body,
        out_shape=(jax.ShapeDtypeStruct((_B, _FP), f32),
                   jax.ShapeDtypeStruct((_B, _FP), f32)),
    )(genomic_feats, fg_indices, Wq_p, bq_p, Wk_p, bk_p, wg_p, Wp_p, bp_p,
      Wf_p, bf_p, fgt_p)

    q1 = _run_edge_layer(h1tab, src, dst, a1p, zeros_hbm)

    h2tab = pl.pallas_call(
        _mid_body,
        out_shape=jax.ShapeDtypeStruct((_N, _ROW), f32),
    )(q1, Wlp, blp, W2p, b2p)

    q2 = _run_edge_layer(h2tab, src, dst, a2p, zeros_hbm)

    out = pl.pallas_call(
        _final_body,
        out_shape=jax.ShapeDtypeStruct((_B, 128), f32),
    )(q2, geno, fg, Wc1a, Wc1b, bc1[None, :], Wo_pad, bo_pad)
    return out[:, :1]


# speedup vs baseline: 1.0795x; 1.0795x over previous
import functools

import jax
import jax.numpy as jnp
from jax import lax
from jax.experimental import pallas as pl
from jax.experimental.pallas import tpu as pltpu
from jax.experimental.pallas import tpu_sc as plsc

_N = 10000
_E = 160000
_HID = 133
_FP = 144
_ROW = 2 * _FP
_B = 1024
_NC = 2
_NS = 16
_NW = _NC * _NS
_EPW = _E // _NW
_CH = 20
_NCHUNK = _EPW // _CH
_NZS = 10
_RPS = _N // _NZS


def _edge_body(h_hbm, src_hbm, dst_hbm, a_hbm, z_hbm, out_hbm,
               idx_s, idx_d, hs0, hd0, hs1, hd1, msg0, msg1, acc,
               gs0, gd0, gs1, gd1, ss0, ss1):
    cid = lax.axis_index("c")
    sid = lax.axis_index("s")
    wid = cid * _NS + sid

    pltpu.sync_copy(a_hbm, msg0.at[pl.ds(0, 2)])
    pltpu.sync_copy(src_hbm.at[wid], idx_s)
    pltpu.sync_copy(dst_hbm.at[wid], idx_d)

    a_regs = [msg0[j // 9, pl.ds(16 * (j % 9), 16)] for j in range(18)]
    lane = lax.iota(jnp.int32, 16)
    perms = [lane ^ k for k in (8, 4, 2, 1)]

    def _allsum(v):
        for p in perms:
            v = v + v.at[p].get(mode="promise_in_bounds")
        return v

    bufs = ((hs0, hd0, msg0, gs0, gd0, ss0), (hs1, hd1, msg1, gs1, gd1, ss1))

    def _issue(c, s):
        hs, hd, msg, gs, gd, ss = bufs[s]
        pltpu.async_copy(h_hbm.at[idx_s.at[c]], hs, gs)
        pltpu.async_copy(h_hbm.at[idx_d.at[c]], hd, gd)

    def _step(c, s):
        hs, hd, msg, gs, gd, ss = bufs[s]
        pltpu.make_async_copy(h_hbm.at[idx_s.at[c]], hs, gs).wait()
        pltpu.make_async_copy(h_hbm.at[idx_d.at[c]], hd, gd).wait()

        @pl.when(c >= 2)
        def _():
            pltpu.make_async_copy(msg, acc.at[idx_d.at[c]], ss).wait()

        @plsc.parallel_loop(0, _CH)
        def edge_body(e):
            acc0 = jnp.zeros((16,), jnp.float32)
            acc1 = jnp.zeros((16,), jnp.float32)
            for j in range(18):
                sv = hs[e, pl.ds(16 * j, 16)]
                dv = hd[e, pl.ds(16 * j, 16)]
                zc = jnp.minimum(sv + dv, 30.0)
                e2 = jnp.exp(zc + zc)
                th = (e2 - 1.0) / (e2 + 1.0)
                contrib = th * a_regs[j]
                if j < 9:
                    acc0 = acc0 + contrib
                else:
                    acc1 = acc1 + contrib
            s0 = _allsum(acc0)
            s1 = _allsum(acc1)
            el0 = jnp.where(s0 > 0.0, s0, jnp.exp(s0) - 1.0)
            el1 = jnp.where(s1 > 0.0, s1, jnp.exp(s1) - 1.0)
            att0 = 0.5 / (1.0 + jnp.exp(el1 - el0))
            att1 = 0.5 - att0
            for j in range(9):
                msg[e, pl.ds(16 * j, 16)] = (att0 * hs[e, pl.ds(16 * j, 16)]
                                             + att1 * hs[e, pl.ds(16 * (j + 9), 16)])
        @pl.when(c + 2 < _NCHUNK)
        def _():
            _issue(c + 2, s)

        pltpu.async_copy(msg, acc.at[idx_d.at[c]], ss, add=True)

    _issue(0, 0)
    _issue(1, 1)

    @pl.when(sid < _NZS)
    def _():
        pltpu.sync_copy(z_hbm, acc.at[pl.ds(sid * _RPS, _RPS)])
    plsc.subcore_barrier()

    def pair_body(g, carry):
        _step(2 * g, 0)
        _step(2 * g + 1, 1)
        return carry

    lax.fori_loop(0, _NCHUNK // 2, pair_body, 0)
    pltpu.make_async_copy(msg0, acc.at[idx_d.at[_NCHUNK - 2]], ss0).wait()
    pltpu.make_async_copy(msg1, acc.at[idx_d.at[_NCHUNK - 1]], ss1).wait()
    plsc.subcore_barrier()

    @pl.when(sid < _NZS)
    def _():
        pltpu.sync_copy(acc.at[pl.ds(sid * _RPS, _RPS)],
                        out_hbm.at[pl.ds(cid * _N + sid * _RPS, _RPS)])


def _run_edge_layer(h_pad, src, dst, a_pad, zeros_hbm):
    mesh = plsc.VectorSubcoreMesh(core_axis_name="c", subcore_axis_name="s",
                                  num_cores=_NC, num_subcores=_NS)
    f = pl.kernel(
        _edge_body,
        out_type=jax.ShapeDtypeStruct((2 * _N, _FP), jnp.float32),
        mesh=mesh,
        scratch_types=[
            pltpu.VMEM((_NCHUNK, _CH), jnp.int32),
            pltpu.VMEM((_NCHUNK, _CH), jnp.int32),
            pltpu.VMEM((_CH, _ROW), jnp.float32),
            pltpu.VMEM((_CH, _ROW), jnp.float32),
            pltpu.VMEM((_CH, _ROW), jnp.float32),
            pltpu.VMEM((_CH, _ROW), jnp.float32),
            pltpu.VMEM((_CH, _FP), jnp.float32),
            pltpu.VMEM((_CH, _FP), jnp.float32),
            pltpu.VMEM_SHARED((_N, _FP), jnp.float32),
            pltpu.SemaphoreType.DMA,
            pltpu.SemaphoreType.DMA,
            pltpu.SemaphoreType.DMA,
            pltpu.SemaphoreType.DMA,
            pltpu.SemaphoreType.DMA,
            pltpu.SemaphoreType.DMA,
        ],
        compiler_params=pltpu.CompilerParams(use_tc_tiling_on_sc=False),
    )
    return f(h_pad, src, dst, a_pad, zeros_hbm)


def _mm1_body(x_ref, w_ref, b_ref, o_ref):
    o_ref[...] = (jnp.dot(x_ref[...], w_ref[...],
                          preferred_element_type=jnp.float32) + b_ref[...])


def _mid_body(p_ref, wl_ref, bl_ref, w2_ref, b2_ref, o_ref):
    hsum = p_ref[0:_N, :] + p_ref[_N:2 * _N, :]
    mid = (jnp.dot(hsum, wl_ref[...], preferred_element_type=jnp.float32)
           + bl_ref[...])
    o_ref[...] = (jnp.dot(mid, w2_ref[...], preferred_element_type=jnp.float32)
                  + b2_ref[...])


def _geno_body(g_ref, fgi_ref, wq_ref, bq_ref, wk_ref, bk_ref, wg_ref,
               wp_ref, bp_ref, wf_ref, bf_ref, fgt_ref, geno_ref, fg_ref):
    f32 = jnp.float32
    g = g_ref[...]
    q = jnp.dot(g, wq_ref[...], preferred_element_type=f32) + bq_ref[...]
    qn = q * (1.0 / jnp.maximum(
        jnp.sqrt(jnp.sum(q * q, axis=1, keepdims=True)), 1e-12))
    kk = jnp.dot(g, wk_ref[...], preferred_element_type=f32) + bk_ref[...]
    kn = kk * (1.0 / jnp.maximum(
        jnp.sqrt(jnp.sum(kk * kk, axis=1, keepdims=True)), 1e-12))
    qw = jnp.sum(qn * wg_ref[...], axis=1, keepdims=True)
    v = qw * (_HID ** -0.5)
    A = v / jnp.maximum(jnp.abs(v), 1e-12)
    G = A * qn
    t1 = (jnp.dot(G * kn, wp_ref[...], preferred_element_type=f32)
          + bp_ref[...] + qn)
    geno_ref[...] = (jnp.dot(t1, wf_ref[...], preferred_element_type=f32)
                     + bf_ref[...])

    vi = lax.broadcasted_iota(jnp.int32, (_B, 200), 1)
    cnt = jnp.zeros((_B, 200), f32)
    for j in range(8):
        cnt = cnt + (fgi_ref[:, j:j + 1] == vi).astype(f32)
    fg_ref[...] = jnp.dot(cnt, fgt_ref[...],
                          preferred_element_type=f32) * 0.125


def _final_body(p_ref, geno_ref, fg_ref, wa_ref, wb_ref,
                bc1_ref, wo_ref, bo_ref, o_ref):
    f32 = jnp.float32
    h2 = p_ref[0:_N, :] + p_ref[_N:2 * _N, :]
    drug_vec = jnp.sum(h2, axis=0, keepdims=True) * (1.0 / _N)
    geno = geno_ref[...]
    drug = drug_vec + fg_ref[...]

    mask = (lax.broadcasted_iota(jnp.int32, (1, _FP), 1) < _HID).astype(f32)
    dmean = jnp.sum(drug, axis=1, keepdims=True) * (1.0 / _HID)
    gmean = jnp.sum(geno, axis=1, keepdims=True) * (1.0 / _HID)
    dm = (drug - dmean) * mask
    gm = (geno - gmean) * mask
    num = jnp.sum(dm * gm, axis=1, keepdims=True)
    den = (jnp.sqrt(jnp.sum(dm * dm, axis=1, keepdims=True))
           * jnp.sqrt(jnp.sum(gm * gm, axis=1, keepdims=True)) + 1e-12)
    boost = 1.0 + jax.nn.sigmoid(num / den)

    hc = jnp.maximum(
        jnp.dot(drug * boost, wa_ref[...], preferred_element_type=f32)
        + jnp.dot(geno * boost, wb_ref[...], preferred_element_type=f32)
        + bc1_ref[...], 0.0)
    o_ref[...] = jax.nn.sigmoid(
        jnp.dot(hc, wo_ref[...], preferred_element_type=f32) + bo_ref[...])


def _pad_heads_cols(W):
    out = jnp.zeros((W.shape[0], _ROW), jnp.float32)
    out = out.at[:, 0:_HID].set(W[:, 0:_HID])
    out = out.at[:, _FP:_FP + _HID].set(W[:, _HID:2 * _HID])
    return out


def kernel(x, genomic_feats, W1, b1, a1, Wl, bl, W2, b2, a2, fg_table, Wq, bq,
           Wk, bk, w_g, Wp, bp, Wf, bf, Wc1, bc1, Wo, bo, edge_index,
           fg_indices):
    f32 = jnp.float32
    src = edge_index[0].reshape(_NW, _NCHUNK, _CH)
    dst = edge_index[1].reshape(_NW, _NCHUNK, _CH)

    W1p = _pad_heads_cols(W1)
    b1p = _pad_heads_cols(b1[None, :])
    a1p = _pad_heads_cols(a1.reshape(1, 2 * _HID)).reshape(2, _FP)
    W2p = jnp.zeros((_FP, _ROW), f32).at[:_HID].set(_pad_heads_cols(W2))
    b2p = _pad_heads_cols(b2[None, :])
    a2p = _pad_heads_cols(a2.reshape(1, 2 * _HID)).reshape(2, _FP)
    Wlp = jnp.zeros((_FP, _FP), f32).at[:_HID, :_HID].set(Wl)
    blp = jnp.zeros((1, _FP), f32).at[0, :_HID].set(bl)
    Wq_p = jnp.zeros((1024, _FP), f32).at[:, :_HID].set(Wq)
    bq_p = jnp.zeros((1, _FP), f32).at[0, :_HID].set(bq)
    Wk_p = jnp.zeros((1024, _FP), f32).at[:, :_HID].set(Wk)
    bk_p = jnp.zeros((1, _FP), f32).at[0, :_HID].set(bk)
    wg_p = jnp.zeros((1, _FP), f32).at[0, :_HID].set(w_g[:, 0])
    Wp_p = jnp.zeros((_FP, _FP), f32).at[:_HID, :_HID].set(Wp)
    bp_p = jnp.zeros((1, _FP), f32).at[0, :_HID].set(bp)
    Wf_p = jnp.zeros((_FP, _FP), f32).at[:_HID, :_HID].set(Wf)
    bf_p = jnp.zeros((1, _FP), f32).at[0, :_HID].set(bf)
    fgt_p = jnp.zeros((200, _FP), f32).at[:, :_HID].set(fg_table)
    Wc1a = jnp.zeros((_FP, 128), f32).at[:_HID].set(Wc1[:_HID])
    Wc1b = jnp.zeros((_FP, 128), f32).at[:_HID].set(Wc1[_HID:])
    Wo_pad = jnp.zeros((128, 128), f32).at[:, :1].set(Wo)
    bo_pad = jnp.zeros((1, 128), f32).at[:, :1].set(bo[None, :])
    zeros_hbm = jnp.zeros((_RPS, _FP), f32)

    h1tab = pl.pallas_call(
        _mm1_body,
        out_shape=jax.ShapeDtypeStruct((_N, _ROW), f32),
    )(x, W1p, b1p)

    geno, fg = pl.pallas_call(
        _geno_body,
        out_shape=(jax.ShapeDtypeStruct((_B, _FP), f32),
                   jax.ShapeDtypeStruct((_B, _FP), f32)),
    )(genomic_feats, fg_indices, Wq_p, bq_p, Wk_p, bk_p, wg_p, Wp_p, bp_p,
      Wf_p, bf_p, fgt_p)

    q1 = _run_edge_layer(h1tab, src, dst, a1p, zeros_hbm)

    h2tab = pl.pallas_call(
        _mid_body,
        out_shape=jax.ShapeDtypeStruct((_N, _ROW), f32),
    )(q1, Wlp, blp, W2p, b2p)

    q2 = _run_edge_layer(h2tab, src, dst, a2p, zeros_hbm)

    out = pl.pallas_call(
        _final_body,
        out_shape=jax.ShapeDtypeStruct((_B, 128), f32),
    )(q2, geno, fg, Wc1a, Wc1b, bc1[None, :], Wo_pad, bo_pad)
    return out[:, :1]
